# baseline (speedup 1.0000x reference)
import functools

import jax
import jax.numpy as jnp
from jax import lax
from jax.experimental import pallas as pl
from jax.experimental.pallas import tpu as pltpu
from jax.experimental.pallas import tpu_sc as plsc

B = 16
N = 10000
E = 160000
D = 9
H = 128
C = B * D
NT = 32
R = 320
NP = NT * R
RB = R + 8
DUMP = R
CH = 8000
NCH = E // CH
VR = CH // 16
FL = CH + 176
CAP = E + 8192
MAGIC = 6554
GCH = 128
HR = 16 * R
HRB = HR + 8

_f32 = jnp.float32
_i32 = jnp.int32


@functools.cache
def _sc_mesh():
    return plsc.VectorSubcoreMesh(core_axis_name="c", subcore_axis_name="s")


_SC_PARAMS = pltpu.CompilerParams(use_tc_tiling_on_sc=False,
                                  needs_layout_passes=False)


def _wid():
    return lax.axis_index("s") * 2 + lax.axis_index("c")


def _prep_body(edges, bdata, bcnt, deg, srcb, dstb, stage, degb, zb, onesb, pkb, locb, cntb):
    wid = _wid()
    base = wid * R
    half_base = lax.axis_index("s") * R
    widv = jnp.broadcast_to(wid, (16,)).astype(_i32)
    iota = lax.iota(_i32, 16)

    def chunk(ch, carry):
        hoff, rem = carry
        pltpu.sync_copy(edges.at[pl.ds(ch * CH, CH)], srcb)
        pltpu.sync_copy(edges.at[pl.ds(E + ch * CH, CH)], dstb)

        def vstep(j, cur_v):
            s = srcb[pl.ds(j * 16, 16)]
            d = dstb[pl.ds(j * 16, 16)]
            p = lax.shift_right_logical(d * MAGIC, 21)
            m = p == widv
            pk = s * 512 + ((d - base) & 511)
            pref = plsc.cumsum(jnp.where(m, 1, 0).astype(_i32))
            plsc.store_scatter(stage, [cur_v + pref - 1], pk, mask=m)
            return cur_v + plsc.all_reduce_population_count(m)

        cur0 = jnp.broadcast_to(rem, (16,)).astype(_i32)
        cur_v = lax.fori_loop(0, VR, vstep, cur0)
        cur_s = jnp.max(cur_v)
        pltpu.sync_copy(stage, bdata.at[wid, pl.ds(pl.multiple_of(hoff, 8), FL)])
        flo = cur_s & (-8)
        tv = plsc.load_gather(stage, [flo + iota])
        plsc.store_scatter(stage, [iota], tv)
        return hoff + flo, cur_s - flo

    hoff, rem = lax.fori_loop(0, NCH, chunk, (_i32(0), _i32(0)))

    dumpv = jnp.broadcast_to(_i32(DUMP), (16,))
    for k in range(32):
        plsc.store_scatter(stage, [rem + iota + 16 * k], dumpv)
    pltpu.sync_copy(stage.at[pl.ds(0, 528)], bdata.at[wid, pl.ds(pl.multiple_of(hoff, 8), 528)])
    total = (hoff + rem + 511) & (-512)

    zf = jnp.zeros((16,), _f32)
    of = jnp.ones((16,), _f32)

    def zrow(i, _):
        zb[i] = zf
        return 0

    lax.fori_loop(0, R, zrow, 0)
    pltpu.sync_copy(zb, degb.at[pl.ds(pl.multiple_of(half_base, R), R)])

    def orow(i, _):
        onesb[i] = of
        return 0

    lax.fori_loop(0, GCH, orow, 0)

    def dchunk(ch, _):
        pltpu.sync_copy(bdata.at[wid, pl.ds(pl.multiple_of(ch * GCH, GCH), GCH)], pkb)
        for j in range(8):
            pkv = pkb[pl.ds(j * 16, 16)]
            loc = pkv & 511
            locb[pl.ds(j * 16, 16)] = jnp.where(loc < R, half_base + loc, HR)
        pltpu.sync_copy(onesb, degb.at[locb], add=True)
        return 0

    lax.fori_loop(0, total // GCH, dchunk, 0)

    pltpu.sync_copy(degb.at[pl.ds(half_base, R)], deg.at[pl.ds(pl.multiple_of(base, R), R)])
    cntb[...] = jnp.where(iota == 0, total, 0).astype(_i32)
    pltpu.sync_copy(cntb, bcnt.at[wid])


@functools.cache
def _prep_kernel():
    return pl.kernel(
        _prep_body,
        out_type=(
            jax.ShapeDtypeStruct((NT, CAP), _i32),
            jax.ShapeDtypeStruct((NT, 16), _i32),
            jax.ShapeDtypeStruct((NP, 16), _f32),
        ),
        mesh=_sc_mesh(),
        compiler_params=_SC_PARAMS,
        scratch_types=(
            pltpu.VMEM((CH,), _i32),
            pltpu.VMEM((CH,), _i32),
            pltpu.VMEM((FL,), _i32),
            pltpu.VMEM_SHARED((HRB, 16), _f32),
            pltpu.VMEM((R, 16), _f32),
            pltpu.VMEM((GCH, 16), _f32),
            pltpu.VMEM((GCH,), _i32),
            pltpu.VMEM((GCH,), _i32),
            pltpu.VMEM((16,), _i32),
        ),
    )


def _prep(edges):
    return _prep_kernel()(edges)


def _agg_body(z, bdata, bcnt, aout, outb, pkb, srcb, locb, rows, cntb,
              spk0, spk1, spk2, spk3, sg0, sg1, sg2, sg3,
              ss0, ss1, ss2, ss3):
    wid = _wid()
    base = wid * R
    half_base = lax.axis_index("s") * R
    row0 = pl.multiple_of(base, R)
    sb0 = pl.multiple_of(half_base, R)
    spk = (spk0, spk1, spk2, spk3)
    sg = (sg0, sg1, sg2, sg3)
    ss = (ss0, ss1, ss2, ss3)
    pltpu.sync_copy(bcnt.at[wid], cntb)
    cnt = jnp.sum(cntb[...])
    nsup = cnt // (4 * GCH)

    for b in (0, 1, 2, 3):
        pltpu.async_copy(bdata.at[wid, pl.ds(b * GCH, GCH)], pkb.at[b],
                         spk[b])
    pltpu.sync_copy(z.at[pl.ds(row0, R)], outb.at[pl.ds(sb0, R)])

    def sup(g, _):
        for b in (0, 1, 2, 3):
            pltpu.make_async_copy(bdata.at[wid, pl.ds(0, GCH)], pkb.at[b],
                                  spk[b]).wait()

            @pl.when(g > 0)
            def _():
                pltpu.make_async_copy(rows.at[b], outb.at[pl.ds(0, GCH)],
                                      ss[b]).wait()

            for j in range(8):
                pkv = pkb[b, pl.ds(j * 16, 16)]
                srcb[b, pl.ds(j * 16, 16)] = lax.shift_right_logical(pkv, 9)
                loc = pkv & 511
                locb[b, pl.ds(j * 16, 16)] = jnp.where(loc < R,
                                                       half_base + loc, HR)
            pltpu.async_copy(z.at[srcb.at[b]], rows.at[b], sg[b])
        for b in (0, 1, 2, 3):
            off = pl.multiple_of((4 * g + b + 4) * GCH, GCH)
            pltpu.async_copy(bdata.at[wid, pl.ds(off, GCH)], pkb.at[b],
                             spk[b])
            pltpu.make_async_copy(z.at[pl.ds(0, GCH)], rows.at[b],
                                  sg[b]).wait()
            pltpu.async_copy(rows.at[b], outb.at[locb.at[b]], ss[b],
                             add=True)
        return 0

    lax.fori_loop(0, nsup, sup, 0)

    @pl.when(nsup > 0)
    def _():
        for b in (0, 1, 2, 3):
            pltpu.make_async_copy(rows.at[b], outb.at[pl.ds(0, GCH)],
                                  ss[b]).wait()

    for b in (0, 1, 2, 3):
        pltpu.make_async_copy(bdata.at[wid, pl.ds(0, GCH)], pkb.at[b],
                              spk[b]).wait()
    pltpu.sync_copy(outb.at[pl.ds(sb0, R)], aout.at[pl.ds(row0, R)])


@functools.cache
def _agg_kernel():
    return pl.kernel(
        _agg_body,
        out_type=jax.ShapeDtypeStruct((NP, C), _f32),
        mesh=_sc_mesh(),
        compiler_params=_SC_PARAMS,
        scratch_types=(
            pltpu.VMEM_SHARED((HRB, C), _f32),
            pltpu.VMEM((4, GCH), _i32),
            pltpu.VMEM((4, GCH), _i32),
            pltpu.VMEM((4, GCH), _i32),
            pltpu.VMEM((4, GCH, C), _f32),
            pltpu.VMEM((16,), _i32),
            pltpu.SemaphoreType.DMA,
            pltpu.SemaphoreType.DMA,
            pltpu.SemaphoreType.DMA,
            pltpu.SemaphoreType.DMA,
            pltpu.SemaphoreType.DMA,
            pltpu.SemaphoreType.DMA,
            pltpu.SemaphoreType.DMA,
            pltpu.SemaphoreType.DMA,
            pltpu.SemaphoreType.DMA,
            pltpu.SemaphoreType.DMA,
            pltpu.SemaphoreType.DMA,
            pltpu.SemaphoreType.DMA,
        ),
    )


def _agg(z, bdata, bcnt):
    return _agg_kernel()(z, bdata, bcnt)


def _k1_body(x_ref, deg_ref, w_ref, o_ref):
    dinv = lax.rsqrt(deg_ref[:, 0:1] + 1.0)
    o_ref[...] = jnp.dot(x_ref[...], w_ref[...],
                         preferred_element_type=_f32) * dinv


def _mid_body(a_ref, deg_ref, bias_ref, w_ref, o_ref):
    dinv = lax.rsqrt(deg_ref[:, 0:1] + 1.0)
    x = jnp.maximum(a_ref[...] * dinv + bias_ref[...], 0.0)
    o_ref[...] = jnp.dot(x, w_ref[...], preferred_element_type=_f32) * dinv


def _k8_body(a_ref, deg_ref, bias_ref, o_ref):
    dinv = lax.rsqrt(deg_ref[:, 0:1] + 1.0)
    o_ref[...] = jnp.tanh(a_ref[...] * dinv + bias_ref[...])


def _k4_body(a_ref, deg_ref, bias_ref, wmu_ref, wvar_ref, bmu_ref, bvar_ref,
             mu_ref, lv_ref, acc_mu, acc_lv):
    i = pl.program_id(0)
    dinv = lax.rsqrt(deg_ref[:, 0:1] + 1.0)
    x = jnp.maximum(a_ref[...] * dinv + bias_ref[...], 0.0)
    cdims = (((0,), (0,)), ((), ()))
    mu_p = jnp.zeros((B, H), _f32)
    lv_p = jnp.zeros((B, H), _f32)
    for d in range(D):
        xd = x[:, d * 16:(d + 1) * 16]
        mu_p = mu_p + lax.dot_general(xd, wmu_ref[d], cdims,
                                      preferred_element_type=_f32)
        lv_p = lv_p + lax.dot_general(xd, wvar_ref[d], cdims,
                                      preferred_element_type=_f32)

    @pl.when(i == 0)
    def _():
        acc_mu[...] = jnp.zeros((B, H), _f32)
        acc_lv[...] = jnp.zeros((B, H), _f32)

    acc_mu[...] += mu_p
    acc_lv[...] += lv_p

    @pl.when(i == 9)
    def _():
        mu_ref[...] = acc_mu[...] + bmu_ref[...]
        lv_ref[...] = acc_lv[...] + bvar_ref[...]


def _k5_body(mu_ref, lv_ref, eps_ref, dwm_ref, w_ref, deg_ref,
             gz_ref, z4_ref):
    gz = mu_ref[...] + eps_ref[...] * jnp.exp(0.5 * lv_ref[...])
    gz_ref[...] = gz
    cols = []
    for d in range(D):
        cols.append(lax.dot_general(dwm_ref[d], gz, (((1,), (1,)), ((), ())),
                                    preferred_element_type=_f32))
    dec = jnp.concatenate(cols, axis=1)
    dinv = lax.rsqrt(deg_ref[:, 0:1] + 1.0)
    z4_ref[...] = jnp.dot(dec, w_ref[...], preferred_element_type=_f32) * dinv


def _rows_spec(rb):
    return pl.BlockSpec((rb, C), lambda i: (i, 0))


def _deg_spec(rb):
    return pl.BlockSpec((rb, 16), lambda i: (i, 0))


_CONST2 = lambda shape: pl.BlockSpec(shape, lambda i: (0, 0))


def _tc_k1(x, deg, wbig):
    return pl.pallas_call(
        _k1_body, grid=(10,),
        in_specs=[_rows_spec(1024), _deg_spec(1024), _CONST2((C, C))],
        out_specs=_rows_spec(1024),
        out_shape=jax.ShapeDtypeStruct((NP, C), _f32),
    )(x, deg, wbig)


def _tc_mid(a, deg, bias, wbig):
    return pl.pallas_call(
        _mid_body, grid=(10,),
        in_specs=[_rows_spec(1024), _deg_spec(1024), _CONST2((1, C)),
                  _CONST2((C, C))],
        out_specs=_rows_spec(1024),
        out_shape=jax.ShapeDtypeStruct((NP, C), _f32),
    )(a, deg, bias, wbig)


def _tc_k8(a, deg, bias):
    return pl.pallas_call(
        _k8_body, grid=(10,),
        in_specs=[_rows_spec(1024), _deg_spec(1024), _CONST2((1, C))],
        out_specs=_rows_spec(1024),
        out_shape=jax.ShapeDtypeStruct((NP, C), _f32),
    )(a, deg, bias)


def _tc_k4(a3, deg, bias, wmu_t, wvar_t, bmu, bvar):
    wspec = pl.BlockSpec((D, 1000, H), lambda i: (0, i, 0))
    return pl.pallas_call(
        _k4_body, grid=(10,),
        in_specs=[_rows_spec(1000), _deg_spec(1000), _CONST2((1, C)),
                  wspec, wspec, _CONST2((1, H)), _CONST2((1, H))],
        out_specs=[_CONST2((B, H)), _CONST2((B, H))],
        out_shape=[jax.ShapeDtypeStruct((B, H), _f32)] * 2,
        scratch_shapes=[pltpu.VMEM((B, H), _f32), pltpu.VMEM((B, H), _f32)],
    )(a3, deg, bias, wmu_t, wvar_t, bmu, bvar)


def _tc_k5(mu, lv, eps, dwm_t, wbig, deg):
    return pl.pallas_call(
        _k5_body, grid=(10,),
        in_specs=[_CONST2((B, H)), _CONST2((B, H)), _CONST2((B, H)),
                  pl.BlockSpec((D, 1024, H), lambda i: (0, i, 0)),
                  _CONST2((C, C)), _deg_spec(1024)],
        out_specs=[_CONST2((B, H)), _rows_spec(1024)],
        out_shape=[jax.ShapeDtypeStruct((B, H), _f32),
                   jax.ShapeDtypeStruct((NP, C), _f32)],
    )(mu, lv, eps, dwm_t, wbig, deg)


def kernel(geo_input, edge_index, ew1, eb1, ew2, eb2, ew3, eb3,
           w_mu, b_mu, w_var, b_var, dw_mlp,
           dw1, db1, dw2, db2, dw3, db3):
    eye16 = jnp.eye(16, dtype=_f32)
    kron = lambda w: jnp.kron(w, eye16)
    bc = lambda b: jnp.repeat(b, 16).reshape(1, C)

    geo_t = jnp.pad(jnp.transpose(geo_input, (1, 2, 0)).reshape(N, C),
                    ((0, NP - N), (0, 0)))
    wmu_t = w_mu.reshape(N, D, H).transpose(1, 0, 2)
    wvar_t = w_var.reshape(N, D, H).transpose(1, 0, 2)
    dwm_t = jnp.pad(dw_mlp.reshape(H, N, D).transpose(2, 1, 0),
                    ((0, 0), (0, NP - N), (0, 0)))
    eps = jax.random.normal(jax.random.key(1), (B, H), _f32)

    bdata, bcnt, deg = _prep(edge_index.reshape(2 * E))

    z1 = _tc_k1(geo_t, deg, kron(ew1))
    a1 = _agg(z1, bdata, bcnt)
    z2 = _tc_mid(a1, deg, bc(eb1), kron(ew2))
    a2 = _agg(z2, bdata, bcnt)
    z3 = _tc_mid(a2, deg, bc(eb2), kron(ew3))
    a3 = _agg(z3, bdata, bcnt)
    mu, logvar = _tc_k4(a3, deg, bc(eb3), wmu_t, wvar_t,
                        b_mu.reshape(1, H), b_var.reshape(1, H))
    geo_z, z4 = _tc_k5(mu, logvar, eps, dwm_t, kron(dw1), deg)
    a4 = _agg(z4, bdata, bcnt)
    z5 = _tc_mid(a4, deg, bc(db1), kron(dw2))
    a5 = _agg(z5, bdata, bcnt)
    z6 = _tc_mid(a5, deg, bc(db2), kron(dw3))
    a6 = _agg(z6, bdata, bcnt)
    y = _tc_k8(a6, deg, bc(db3))

    geo_output = y[:N].reshape(N, D, B).transpose(2, 0, 1)
    return (geo_z, geo_output, mu, logvar)


# speedup vs baseline: 1.6388x; 1.6388x over previous
import functools

import jax
import jax.numpy as jnp
from jax import lax
from jax.experimental import pallas as pl
from jax.experimental.pallas import tpu as pltpu
from jax.experimental.pallas import tpu_sc as plsc

B = 16
N = 10000
E = 160000
D = 9
H = 128
C = B * D
NT = 32
R = 320
NP = NT * R
RB = R + 8
DUMP = R
CH = 8000
NCH = E // CH
VR = CH // 16
FL = CH + 176
CAP = E + 8192
MAGIC = 6554
GCH = 128
HR = 16 * R
HRB = HR + 8

_f32 = jnp.float32
_i32 = jnp.int32


@functools.cache
def _sc_mesh():
    return plsc.VectorSubcoreMesh(core_axis_name="c", subcore_axis_name="s")


_SC_PARAMS = pltpu.CompilerParams(use_tc_tiling_on_sc=False,
                                  needs_layout_passes=False)


def _wid():
    return lax.axis_index("s") * 2 + lax.axis_index("c")


def _prep_body(edges, bdata, bcnt, deg, srcb, dstb, stage, degb, zb, onesb, pkb, locb, cntb):
    wid = _wid()
    base = wid * R
    half_base = lax.axis_index("s") * R
    widv = jnp.broadcast_to(wid, (16,)).astype(_i32)
    iota = lax.iota(_i32, 16)

    def chunk(ch, carry):
        hoff, rem = carry
        pltpu.sync_copy(edges.at[pl.ds(ch * CH, CH)], srcb)
        pltpu.sync_copy(edges.at[pl.ds(E + ch * CH, CH)], dstb)

        def vstep(j, cur_v):
            s = srcb[pl.ds(j * 16, 16)]
            d = dstb[pl.ds(j * 16, 16)]
            p = lax.shift_right_logical(d * MAGIC, 21)
            m = p == widv
            pk = s * 512 + ((d - base) & 511)
            pref = plsc.cumsum(jnp.where(m, 1, 0).astype(_i32))
            plsc.store_scatter(stage, [cur_v + pref - 1], pk, mask=m)
            return cur_v + plsc.all_reduce_population_count(m)

        cur0 = jnp.broadcast_to(rem, (16,)).astype(_i32)
        cur_v = lax.fori_loop(0, VR, vstep, cur0)
        cur_s = jnp.max(cur_v)
        pltpu.sync_copy(stage, bdata.at[wid, pl.ds(pl.multiple_of(hoff, 8), FL)])
        flo = cur_s & (-8)
        tv = plsc.load_gather(stage, [flo + iota])
        plsc.store_scatter(stage, [iota], tv)
        return hoff + flo, cur_s - flo

    hoff, rem = lax.fori_loop(0, NCH, chunk, (_i32(0), _i32(0)))

    dumpv = jnp.broadcast_to(_i32(DUMP), (16,))
    for k in range(8):
        plsc.store_scatter(stage, [rem + iota + 16 * k], dumpv)
    pltpu.sync_copy(stage.at[pl.ds(0, 144)], bdata.at[wid, pl.ds(pl.multiple_of(hoff, 8), 144)])
    total = (hoff + rem + 127) & (-128)

    zf = jnp.zeros((16,), _f32)
    of = jnp.ones((16,), _f32)

    def zrow(i, _):
        zb[i] = zf
        return 0

    lax.fori_loop(0, R, zrow, 0)
    pltpu.sync_copy(zb, degb.at[pl.ds(pl.multiple_of(half_base, R), R)])

    def orow(i, _):
        onesb[i] = of
        return 0

    lax.fori_loop(0, GCH, orow, 0)

    def dchunk(ch, _):
        pltpu.sync_copy(bdata.at[wid, pl.ds(pl.multiple_of(ch * GCH, GCH), GCH)], pkb)
        for j in range(8):
            pkv = pkb[pl.ds(j * 16, 16)]
            loc = pkv & 511
            locb[pl.ds(j * 16, 16)] = jnp.where(loc < R, half_base + loc, HR)
        pltpu.sync_copy(onesb, degb.at[locb], add=True)
        return 0

    lax.fori_loop(0, total // GCH, dchunk, 0)

    pltpu.sync_copy(degb.at[pl.ds(half_base, R)], deg.at[pl.ds(pl.multiple_of(base, R), R)])
    cntb[...] = jnp.where(iota == 0, total, 0).astype(_i32)
    pltpu.sync_copy(cntb, bcnt.at[wid])


@functools.cache
def _prep_kernel():
    return pl.kernel(
        _prep_body,
        out_type=(
            jax.ShapeDtypeStruct((NT, CAP), _i32),
            jax.ShapeDtypeStruct((NT, 16), _i32),
            jax.ShapeDtypeStruct((NP, 16), _f32),
        ),
        mesh=_sc_mesh(),
        compiler_params=_SC_PARAMS,
        scratch_types=(
            pltpu.VMEM((CH,), _i32),
            pltpu.VMEM((CH,), _i32),
            pltpu.VMEM((FL,), _i32),
            pltpu.VMEM_SHARED((HRB, 16), _f32),
            pltpu.VMEM((R, 16), _f32),
            pltpu.VMEM((GCH, 16), _f32),
            pltpu.VMEM((GCH,), _i32),
            pltpu.VMEM((GCH,), _i32),
            pltpu.VMEM((16,), _i32),
        ),
    )


def _prep(edges):
    return _prep_kernel()(edges)


def _agg_body(z, bdata, bcnt, aout, outb, pkb, srcb, locb, rows, cntb, sem):
    wid = _wid()
    base = wid * R
    half_base = lax.axis_index("s") * R
    pltpu.sync_copy(bcnt.at[wid], cntb)
    cnt = jnp.sum(cntb[...])
    pltpu.sync_copy(z.at[pl.ds(pl.multiple_of(base, R), R)], outb.at[pl.ds(pl.multiple_of(half_base, R), R)])

    def chunk(ch, _):
        pltpu.sync_copy(bdata.at[wid, pl.ds(pl.multiple_of(ch * GCH, GCH), GCH)], pkb)
        for j in range(8):
            pkv = pkb[pl.ds(j * 16, 16)]
            srcb[pl.ds(j * 16, 16)] = lax.shift_right_logical(pkv, 9)
            loc = pkv & 511
            locb[pl.ds(j * 16, 16)] = jnp.where(loc < R, half_base + loc, HR)
        pltpu.async_copy(z.at[srcb], rows, sem).wait()
        pltpu.sync_copy(rows, outb.at[locb], add=True)
        return 0

    lax.fori_loop(0, cnt // GCH, chunk, 0)
    pltpu.sync_copy(outb.at[pl.ds(pl.multiple_of(half_base, R), R)], aout.at[pl.ds(pl.multiple_of(base, R), R)])


@functools.cache
def _agg_kernel():
    return pl.kernel(
        _agg_body,
        out_type=jax.ShapeDtypeStruct((NP, C), _f32),
        mesh=_sc_mesh(),
        compiler_params=_SC_PARAMS,
        scratch_types=(
            pltpu.VMEM_SHARED((HRB, C), _f32),
            pltpu.VMEM((GCH,), _i32),
            pltpu.VMEM((GCH,), _i32),
            pltpu.VMEM((GCH,), _i32),
            pltpu.VMEM((GCH, C), _f32),
            pltpu.VMEM((16,), _i32),
            pltpu.SemaphoreType.DMA,
        ),
    )


def _agg(z, bdata, bcnt):
    return _agg_kernel()(z, bdata, bcnt)


def _k1_body(x_ref, deg_ref, w_ref, o_ref):
    dinv = lax.rsqrt(deg_ref[:, 0:1] + 1.0)
    o_ref[...] = jnp.dot(x_ref[...], w_ref[...],
                         preferred_element_type=_f32) * dinv


def _mid_body(a_ref, deg_ref, bias_ref, w_ref, o_ref):
    dinv = lax.rsqrt(deg_ref[:, 0:1] + 1.0)
    x = jnp.maximum(a_ref[...] * dinv + bias_ref[...], 0.0)
    o_ref[...] = jnp.dot(x, w_ref[...], preferred_element_type=_f32) * dinv


def _k8_body(a_ref, deg_ref, bias_ref, o_ref):
    dinv = lax.rsqrt(deg_ref[:, 0:1] + 1.0)
    o_ref[...] = jnp.tanh(a_ref[...] * dinv + bias_ref[...])


def _k4_body(a_ref, deg_ref, bias_ref, wmu_ref, wvar_ref, bmu_ref, bvar_ref,
             mu_ref, lv_ref, acc_mu, acc_lv):
    i = pl.program_id(0)
    dinv = lax.rsqrt(deg_ref[:, 0:1] + 1.0)
    x = jnp.maximum(a_ref[...] * dinv + bias_ref[...], 0.0)
    cdims = (((0,), (0,)), ((), ()))
    mu_p = jnp.zeros((B, H), _f32)
    lv_p = jnp.zeros((B, H), _f32)
    for d in range(D):
        xd = x[:, d * 16:(d + 1) * 16]
        mu_p = mu_p + lax.dot_general(xd, wmu_ref[:, d, :], cdims,
                                      preferred_element_type=_f32)
        lv_p = lv_p + lax.dot_general(xd, wvar_ref[:, d, :], cdims,
                                      preferred_element_type=_f32)

    @pl.when(i == 0)
    def _():
        acc_mu[...] = jnp.zeros((B, H), _f32)
        acc_lv[...] = jnp.zeros((B, H), _f32)

    acc_mu[...] += mu_p
    acc_lv[...] += lv_p

    @pl.when(i == 9)
    def _():
        mu_ref[...] = acc_mu[...] + bmu_ref[...]
        lv_ref[...] = acc_lv[...] + bvar_ref[...]


def _k5_body(mu_ref, lv_ref, eps_ref, dwm_ref, w_ref, deg_ref,
             gz_ref, z4_ref):
    gz = mu_ref[...] + eps_ref[...] * jnp.exp(0.5 * lv_ref[...])
    gz_ref[...] = gz
    cols = []
    for d in range(D):
        cols.append(lax.dot_general(dwm_ref[:, d, :], gz,
                                    (((1,), (1,)), ((), ())),
                                    preferred_element_type=_f32))
    dec = jnp.concatenate(cols, axis=1)
    dinv = lax.rsqrt(deg_ref[:, 0:1] + 1.0)
    z4_ref[...] = jnp.dot(dec, w_ref[...], preferred_element_type=_f32) * dinv


def _rows_spec(rb):
    return pl.BlockSpec((rb, C), lambda i: (i, 0))


def _deg_spec(rb):
    return pl.BlockSpec((rb, 16), lambda i: (i, 0))


_CONST2 = lambda shape: pl.BlockSpec(shape, lambda i: (0, 0))


def _tc_k1(x, deg, wbig):
    return pl.pallas_call(
        _k1_body, grid=(10,),
        in_specs=[_rows_spec(1024), _deg_spec(1024), _CONST2((C, C))],
        out_specs=_rows_spec(1024),
        out_shape=jax.ShapeDtypeStruct((NP, C), _f32),
    )(x, deg, wbig)


def _tc_mid(a, deg, bias, wbig):
    return pl.pallas_call(
        _mid_body, grid=(10,),
        in_specs=[_rows_spec(1024), _deg_spec(1024), _CONST2((1, C)),
                  _CONST2((C, C))],
        out_specs=_rows_spec(1024),
        out_shape=jax.ShapeDtypeStruct((NP, C), _f32),
    )(a, deg, bias, wbig)


def _tc_k8(a, deg, bias):
    return pl.pallas_call(
        _k8_body, grid=(10,),
        in_specs=[_rows_spec(1024), _deg_spec(1024), _CONST2((1, C))],
        out_specs=_rows_spec(1024),
        out_shape=jax.ShapeDtypeStruct((NP, C), _f32),
    )(a, deg, bias)


def _tc_k4(a3, deg, bias, w_mu, w_var, bmu, bvar):
    wspec = pl.BlockSpec((1000, D, H), lambda i: (i, 0, 0))
    return pl.pallas_call(
        _k4_body, grid=(10,),
        in_specs=[_rows_spec(1000), _deg_spec(1000), _CONST2((1, C)),
                  wspec, wspec, _CONST2((1, H)), _CONST2((1, H))],
        out_specs=[_CONST2((B, H)), _CONST2((B, H))],
        out_shape=[jax.ShapeDtypeStruct((B, H), _f32)] * 2,
        scratch_shapes=[pltpu.VMEM((B, H), _f32), pltpu.VMEM((B, H), _f32)],
    )(a3, deg, bias, w_mu.reshape(N, D, H), w_var.reshape(N, D, H),
      bmu, bvar)


def _tc_k5(mu, lv, eps, dw_mlp, wbig, deg):
    return pl.pallas_call(
        _k5_body, grid=(10,),
        in_specs=[_CONST2((B, H)), _CONST2((B, H)), _CONST2((B, H)),
                  pl.BlockSpec((1000, D, H), lambda i: (i, 0, 0)),
                  _CONST2((C, C)), _deg_spec(1000)],
        out_specs=[_CONST2((B, H)), _rows_spec(1000)],
        out_shape=[jax.ShapeDtypeStruct((B, H), _f32),
                   jax.ShapeDtypeStruct((NP, C), _f32)],
    )(mu, lv, eps, dw_mlp, wbig, deg)


def kernel(geo_input, edge_index, ew1, eb1, ew2, eb2, ew3, eb3,
           w_mu, b_mu, w_var, b_var, dw_mlp,
           dw1, db1, dw2, db2, dw3, db3):
    eye16 = jnp.eye(16, dtype=_f32)
    kron = lambda w: jnp.kron(w, eye16)
    bc = lambda b: jnp.repeat(b, 16).reshape(1, C)

    geo_t = jnp.pad(jnp.transpose(geo_input, (1, 2, 0)).reshape(N, C),
                    ((0, NP - N), (0, 0)))
    eps = jax.random.normal(jax.random.key(1), (B, H), _f32)

    bdata, bcnt, deg = _prep(edge_index.reshape(2 * E))

    z1 = _tc_k1(geo_t, deg, kron(ew1))
    a1 = _agg(z1, bdata, bcnt)
    z2 = _tc_mid(a1, deg, bc(eb1), kron(ew2))
    a2 = _agg(z2, bdata, bcnt)
    z3 = _tc_mid(a2, deg, bc(eb2), kron(ew3))
    a3 = _agg(z3, bdata, bcnt)
    mu, logvar = _tc_k4(a3, deg, bc(eb3), w_mu, w_var,
                        b_mu.reshape(1, H), b_var.reshape(1, H))
    geo_z, z4 = _tc_k5(mu, logvar, eps, dw_mlp.T.reshape(N, D, H),
                       kron(dw1), deg)
    a4 = _agg(z4, bdata, bcnt)
    z5 = _tc_mid(a4, deg, bc(db1), kron(dw2))
    a5 = _agg(z5, bdata, bcnt)
    z6 = _tc_mid(a5, deg, bc(db2), kron(dw3))
    a6 = _agg(z6, bdata, bcnt)
    y = _tc_k8(a6, deg, bc(db3))

    geo_output = y[:N].reshape(N, D, B).transpose(2, 0, 1)
    return (geo_z, geo_output, mu, logvar)
